# Initial kernel scaffold; baseline (speedup 1.0000x reference)
import functools

import jax
import jax.numpy as jnp
from jax import lax
from jax.experimental import pallas as pl
from jax.experimental.pallas import tpu as pltpu
from jax.experimental.pallas import tpu_sc as plsc

N_NODES = 10000
N_EDGES = 320000
IN_DIM = 128
HIDDEN = 128
NUM_CLASSES = 64

NC = 2
NS = 16
NW = NC * NS
CHUNK = 128
ROWS_PER_TILE = N_NODES // NS
PAD_ROWS = ROWS_PER_TILE + 1
N_PAD = PAD_ROWS * NS
DUMMY = N_NODES
CPW = -(-N_EDGES // (CHUNK * NW))
N_CHUNKS = CPW * NW
E_PAD = N_CHUNKS * CHUNK
EPS = 1e-5

_MESH = plsc.VectorSubcoreMesh(
    core_axis_name="c", subcore_axis_name="s", num_cores=NC, num_subcores=NS)


@functools.partial(
    pl.kernel,
    out_type=[
        jax.ShapeDtypeStruct((NC, N_NODES, 16), jnp.float32),
        jax.ShapeDtypeStruct((NC, N_NODES, 16), jnp.float32),
    ],
    mesh=_MESH,
    scratch_types=[
        pltpu.VMEM((CHUNK,), jnp.int32),
        pltpu.VMEM((CHUNK, 16), jnp.float32),
        pltpu.VMEM((PAD_ROWS, 16), jnp.float32),
        pltpu.VMEM_SHARED((N_PAD, 16), jnp.float32),
        pltpu.VMEM_SHARED((N_PAD, 16), jnp.float32),
    ],
)
def _sc_degrees(src_hbm, dst_hbm, odeg_out, ideg_out,
                idx_v, ones_v, zeros_v, acc_o, acc_i):
    c = lax.axis_index("c")
    s = lax.axis_index("s")
    wid = s * NC + c

    def fill(i, _):
        ones_v[i, :] = jnp.full((16,), 1.0, jnp.float32)
        zeros_v[i, :] = jnp.zeros((16,), jnp.float32)
        return 0

    lax.fori_loop(0, PAD_ROWS, fill, 0)

    def fill_ones_tail(i, _):
        ones_v[i, :] = jnp.full((16,), 1.0, jnp.float32)
        return 0

    lax.fori_loop(PAD_ROWS, CHUNK, fill_ones_tail, 0)
    pltpu.sync_copy(zeros_v, acc_o.at[pl.ds(s * PAD_ROWS, PAD_ROWS)])
    pltpu.sync_copy(zeros_v, acc_i.at[pl.ds(s * PAD_ROWS, PAD_ROWS)])
    plsc.subcore_barrier()

    def step(j, _):
        chunk = wid * CPW + j
        pltpu.sync_copy(src_hbm.at[chunk], idx_v)
        pltpu.sync_copy(ones_v, acc_o.at[idx_v], add=True)
        pltpu.sync_copy(dst_hbm.at[chunk], idx_v)
        pltpu.sync_copy(ones_v, acc_i.at[idx_v], add=True)
        return 0

    lax.fori_loop(0, CPW, step, 0)
    plsc.subcore_barrier()
    pltpu.sync_copy(acc_o.at[pl.ds(s * ROWS_PER_TILE, ROWS_PER_TILE)],
                    odeg_out.at[c, pl.ds(s * ROWS_PER_TILE, ROWS_PER_TILE)])
    pltpu.sync_copy(acc_i.at[pl.ds(s * ROWS_PER_TILE, ROWS_PER_TILE)],
                    ideg_out.at[c, pl.ds(s * ROWS_PER_TILE, ROWS_PER_TILE)])


def _make_sc_aggregate(D):
    @functools.partial(
        pl.kernel,
        out_type=jax.ShapeDtypeStruct((NC, N_NODES, D), jnp.float32),
        mesh=_MESH,
        scratch_types=[
            pltpu.VMEM((CHUNK,), jnp.int32),
            pltpu.VMEM((CHUNK,), jnp.int32),
            pltpu.VMEM((CHUNK, D), jnp.float32),
            pltpu.VMEM((PAD_ROWS, D), jnp.float32),
            pltpu.VMEM_SHARED((N_PAD, D), jnp.float32),
            pltpu.SemaphoreType.DMA,
        ],
    )
    def agg(g_hbm, src_hbm, dst_hbm, out_hbm,
            sidx, didx, rows, zeros_v, acc, sem):
        c = lax.axis_index("c")
        s = lax.axis_index("s")
        wid = s * NC + c

        def fillz(i, _):
            for k in range(D // 16):
                zeros_v[i, pl.ds(16 * k, 16)] = jnp.zeros((16,), jnp.float32)
            return 0

        lax.fori_loop(0, PAD_ROWS, fillz, 0)
        pltpu.sync_copy(zeros_v, acc.at[pl.ds(s * PAD_ROWS, PAD_ROWS)])
        plsc.subcore_barrier()

        def step(j, _):
            chunk = wid * CPW + j
            pltpu.sync_copy(src_hbm.at[chunk], sidx)
            pltpu.sync_copy(dst_hbm.at[chunk], didx)
            pltpu.async_copy(g_hbm.at[sidx], rows, sem).wait()
            pltpu.sync_copy(rows, acc.at[didx], add=True)
            return 0

        lax.fori_loop(0, CPW, step, 0)
        plsc.subcore_barrier()
        pltpu.sync_copy(acc.at[pl.ds(s * ROWS_PER_TILE, ROWS_PER_TILE)],
                        out_hbm.at[c, pl.ds(s * ROWS_PER_TILE, ROWS_PER_TILE)])

    return agg


_sc_agg128 = _make_sc_aggregate(HIDDEN)
_sc_agg64 = _make_sc_aggregate(NUM_CLASSES)


def _out_norm_col(od_ref):
    d = od_ref[0, :, 0:1] + od_ref[1, :, 0:1]
    return jnp.where(d > 0, lax.rsqrt(d), 0.0)


def _tc_prescale_body(od_ref, f_ref, o_ref):
    o_ref[0:N_NODES, :] = f_ref[...] * _out_norm_col(od_ref)
    o_ref[N_NODES:N_PAD, :] = jnp.zeros((N_PAD - N_NODES, IN_DIM), jnp.float32)


_tc_prescale = pl.pallas_call(
    _tc_prescale_body,
    out_shape=jax.ShapeDtypeStruct((N_PAD, IN_DIM), jnp.float32))


def _make_tc_layer(fold_next):
    Do = NUM_CLASSES if fold_next else HIDDEN

    def body(p_ref, od_ref, id_ref, W_ref, b_ref, *rest):
        if fold_next:
            Wn_ref, o_ref = rest
        else:
            (o_ref,) = rest
        in_norm = _out_norm_col(id_ref)
        agg = (p_ref[0, 0:N_NODES, :] + p_ref[1, 0:N_NODES, :]) * in_norm
        h = jnp.dot(agg, W_ref[...], preferred_element_type=jnp.float32)
        h = jnp.maximum(h + b_ref[...], 0.0)
        mu = jnp.mean(h, axis=0, keepdims=True)
        var = jnp.mean((h - mu) ** 2, axis=0, keepdims=True)
        g = (h - mu) * lax.rsqrt(var + EPS) * _out_norm_col(od_ref)
        if fold_next:
            g = jnp.dot(g, Wn_ref[...], preferred_element_type=jnp.float32)
        o_ref[0:N_NODES, :] = g
        o_ref[N_NODES:N_PAD, :] = jnp.zeros((N_PAD - N_NODES, Do), jnp.float32)

    return pl.pallas_call(
        body, out_shape=jax.ShapeDtypeStruct((N_PAD, Do), jnp.float32))


_tc_layer0 = _make_tc_layer(fold_next=False)
_tc_layer1 = _make_tc_layer(fold_next=True)


def _tc_final_body(p_ref, id_ref, b_ref, o_ref):
    in_norm = _out_norm_col(id_ref)
    o_ref[...] = ((p_ref[0, 0:N_NODES, :] + p_ref[1, 0:N_NODES, :]) * in_norm
                  + b_ref[...])


_tc_final = pl.pallas_call(
    _tc_final_body,
    out_shape=jax.ShapeDtypeStruct((N_NODES, NUM_CLASSES), jnp.float32))


def kernel(features, edge_index, W0, b0, W1, b1, W2, b2):
    src = edge_index[0].astype(jnp.int32)
    dst = edge_index[1].astype(jnp.int32)
    pad = jnp.full((E_PAD - N_EDGES,), DUMMY, jnp.int32)
    src_p = jnp.concatenate([src, pad]).reshape(N_CHUNKS, CHUNK)
    dst_p = jnp.concatenate([dst, pad]).reshape(N_CHUNKS, CHUNK)

    odeg, ideg = _sc_degrees(src_p, dst_p)
    g0 = _tc_prescale(odeg, features)
    p0 = _sc_agg128(g0, src_p, dst_p)
    g1 = _tc_layer0(p0, odeg, ideg, W0, b0.reshape(1, HIDDEN))
    p1 = _sc_agg128(g1, src_p, dst_p)
    g2 = _tc_layer1(p1, odeg, ideg, W1, b1.reshape(1, HIDDEN), W2)
    p2 = _sc_agg64(g2, src_p, dst_p)
    return _tc_final(p2, ideg, b2.reshape(1, NUM_CLASSES))


# speedup vs baseline: 1.6227x; 1.6227x over previous
import functools

import jax
import jax.numpy as jnp
from jax import lax
from jax.experimental import pallas as pl
from jax.experimental.pallas import tpu as pltpu
from jax.experimental.pallas import tpu_sc as plsc

N_NODES = 10000
N_EDGES = 320000
IN_DIM = 128
HIDDEN = 128
NUM_CLASSES = 64
D = 128

NC = 2
NS = 16
NW = NC * NS
CHUNK = 128
PAD_ROWS = 632
N_PAD = PAD_ROWS * NS
DUMMY = N_NODES
CPW = -(-N_EDGES // (CHUNK * NW))
N_CHUNKS = CPW * NW
E_PAD = N_CHUNKS * CHUNK
EPS = 1e-5

_MESH = plsc.VectorSubcoreMesh(
    core_axis_name="c", subcore_axis_name="s", num_cores=NC, num_subcores=NS)


CORE_ROWS = N_PAD // NC
ACC_ROWS = 5120
ZPT = ACC_ROWS // NS
OPT = CORE_ROWS // (NS // 2)
CPT = N_CHUNKS // NS


@functools.partial(
    pl.kernel,
    out_type=jax.ShapeDtypeStruct((N_PAD, D), jnp.float32),
    mesh=_MESH,
    scratch_types=[
        pltpu.VMEM((1, CHUNK), jnp.int32),
        pltpu.VMEM((1, CHUNK), jnp.int32),
        pltpu.VMEM((1, CHUNK), jnp.int32),
        pltpu.VMEM((CHUNK, D), jnp.float32),
        pltpu.VMEM((ZPT, D), jnp.float32),
        pltpu.VMEM_SHARED((ACC_ROWS, D), jnp.float32),
        pltpu.SemaphoreType.DMA,
    ],
)
def _sc_agg(g_hbm, src_hbm, dst_hbm, out_hbm,
            sidx, didx, ldidx, rows, zeros_v, acc, sem):
    c = lax.axis_index("c")
    s = lax.axis_index("s")
    base = c * CORE_ROWS

    def fillz(i, _):
        for k in range(D // 16):
            zeros_v[i, pl.ds(16 * k, 16)] = jnp.zeros((16,), jnp.float32)
        return 0

    lax.fori_loop(0, ZPT, fillz, 0)
    pltpu.sync_copy(zeros_v, acc.at[pl.ds(s * ZPT, ZPT)])
    plsc.subcore_barrier()

    def step(j, _):
        chunk = s * CPT + j
        pltpu.sync_copy(src_hbm.at[chunk], sidx)
        pltpu.sync_copy(dst_hbm.at[chunk], didx)
        for k in range(CHUNK // 16):
            t = didx[0, pl.ds(16 * k, 16)] - base
            ok = (t >= 0) & (t < CORE_ROWS)
            ldidx[0, pl.ds(16 * k, 16)] = jnp.where(ok, t, CORE_ROWS)
        pltpu.async_copy(g_hbm.at[sidx.at[0]], rows, sem).wait()
        pltpu.sync_copy(rows, acc.at[ldidx.at[0]], add=True)
        return 0

    lax.fori_loop(0, CPT, step, 0)
    plsc.subcore_barrier()

    @pl.when(s < NS // 2)
    def _copy_out():
        pltpu.sync_copy(acc.at[pl.ds(s * OPT, OPT)],
                        out_hbm.at[pl.ds(base + s * OPT, OPT)])


def _norm_col(deg_ref):
    d = deg_ref[0:N_NODES, 0:1]
    return jnp.where(d > 0, lax.rsqrt(d), 0.0)


def _tc_prescale_body(od_ref, f_ref, o_ref):
    o_ref[0:N_NODES, :] = f_ref[...] * _norm_col(od_ref)
    o_ref[N_NODES:N_PAD, :] = jnp.zeros((N_PAD - N_NODES, IN_DIM), jnp.float32)


_tc_prescale = pl.pallas_call(
    _tc_prescale_body,
    out_shape=jax.ShapeDtypeStruct((N_PAD, IN_DIM), jnp.float32))


def _tc_layer_body(p_ref, od_ref, id_ref, W_ref, b_ref, o_ref):
    agg = p_ref[0:N_NODES, :] * _norm_col(id_ref)
    h = jnp.dot(agg, W_ref[...], preferred_element_type=jnp.float32)
    h = jnp.maximum(h + b_ref[...], 0.0)
    mu = jnp.mean(h, axis=0, keepdims=True)
    var = jnp.mean((h - mu) ** 2, axis=0, keepdims=True)
    g = (h - mu) * lax.rsqrt(var + EPS) * _norm_col(od_ref)
    o_ref[0:N_NODES, :] = g
    o_ref[N_NODES:N_PAD, :] = jnp.zeros((N_PAD - N_NODES, HIDDEN), jnp.float32)


_tc_layer = pl.pallas_call(
    _tc_layer_body,
    out_shape=jax.ShapeDtypeStruct((N_PAD, HIDDEN), jnp.float32))


def _tc_final_body(p_ref, id_ref, W_ref, b_ref, o_ref):
    agg = p_ref[0:N_NODES, :] * _norm_col(id_ref)
    o_ref[...] = jnp.dot(agg, W_ref[...],
                         preferred_element_type=jnp.float32) + b_ref[...]


_tc_final = pl.pallas_call(
    _tc_final_body,
    out_shape=jax.ShapeDtypeStruct((N_NODES, NUM_CLASSES), jnp.float32))


def kernel(features, edge_index, W0, b0, W1, b1, W2, b2):
    src = edge_index[0].astype(jnp.int32)
    dst = edge_index[1].astype(jnp.int32)
    pad = jnp.full((E_PAD - N_EDGES,), DUMMY, jnp.int32)
    src_p = jnp.concatenate([src, pad]).reshape(N_CHUNKS, 1, CHUNK)
    dst_p = jnp.concatenate([dst, pad]).reshape(N_CHUNKS, 1, CHUNK)
    ones = jnp.ones((N_PAD, D), jnp.float32)

    odeg = _sc_agg(ones, src_p, src_p)
    ones_b, dst_b, odeg = lax.optimization_barrier((ones, dst_p, odeg))
    ideg = _sc_agg(ones_b, dst_b, dst_b)
    g0 = _tc_prescale(odeg, features)
    p0 = _sc_agg(g0, src_p, dst_p)
    g1 = _tc_layer(p0, odeg, ideg, W0, b0.reshape(1, HIDDEN))
    p1 = _sc_agg(g1, src_p, dst_p)
    g2 = _tc_layer(p1, odeg, ideg, W1, b1.reshape(1, HIDDEN))
    p2 = _sc_agg(g2, src_p, dst_p)
    return _tc_final(p2, ideg, W2, b2.reshape(1, NUM_CLASSES))
